# baseline (speedup 1.0000x reference)
import jax
import jax.numpy as jnp
from jax.experimental import pallas as pl


def _add_body(x_ref, e_ref, o_ref):
    o_ref[...] = x_ref[...] + e_ref[...][None]


def kernel(x, emb_table):
    B, T, D = x.shape
    BT = 256
    return pl.pallas_call(
        _add_body,
        grid=(T // BT,),
        in_specs=[
            pl.BlockSpec((B, BT, D), lambda i: (0, i, 0)),
            pl.BlockSpec((BT, D), lambda i: (i, 0)),
        ],
        out_specs=pl.BlockSpec((B, BT, D), lambda i: (0, i, 0)),
        out_shape=jax.ShapeDtypeStruct(x.shape, x.dtype),
    )(x, emb_table)


# speedup vs baseline: 1.0112x; 1.0112x over previous
import jax
import jax.numpy as jnp
from jax.experimental import pallas as pl


def _add_body(x_ref, e_ref, o_ref):
    o_ref[...] = x_ref[...] + e_ref[...][None]


def kernel(x, emb_table):
    B, T, D = x.shape
    BT = 1024
    BB = 2
    return pl.pallas_call(
        _add_body,
        grid=(T // BT, B // BB),
        in_specs=[
            pl.BlockSpec((BB, BT, D), lambda i, j: (j, i, 0)),
            pl.BlockSpec((BT, D), lambda i, j: (i, 0)),
        ],
        out_specs=pl.BlockSpec((BB, BT, D), lambda i, j: (j, i, 0)),
        out_shape=jax.ShapeDtypeStruct(x.shape, x.dtype),
    )(x, emb_table)
